# baseline (speedup 1.0000x reference)
import functools

import jax
import jax.numpy as jnp
from jax import lax
from jax.experimental import pallas as pl
from jax.experimental.pallas import tpu as pltpu
from jax.experimental.pallas import tpu_sc as plsc

N = 10000
E = 320000
F1 = 128
CP = 48
CLASSES = 40

NC, NS = 2, 16
NW = NC * NS
K = 128
NCHUNK = 80
E_PAD = NW * NCHUNK * K
ACC_ROWS = 10112
STRIPE = ACC_ROWS // NS

BM = 1000


def _sc_mesh():
    return plsc.VectorSubcoreMesh(
        core_axis_name="c", subcore_axis_name="s",
        num_cores=NC, num_subcores=NS)


@functools.partial(
    pl.kernel,
    out_type=jax.ShapeDtypeStruct((NC, ACC_ROWS, 16), jnp.float32),
    mesh=_sc_mesh(),
    scratch_types=[
        pltpu.VMEM((NCHUNK, K), jnp.int32),
        pltpu.VMEM((K, 16), jnp.float32),
        pltpu.VMEM_SHARED((ACC_ROWS, 16), jnp.float32),
    ],
    compiler_params=pltpu.CompilerParams(use_tc_tiling_on_sc=False),
)
def _deg_kernel(dst_hbm, ones_hbm, zeros_hbm, out_hbm, dst_v, ones_v, acc_s):
    c = lax.axis_index("c")
    s = lax.axis_index("s")
    wid = c * NS + s
    pltpu.sync_copy(zeros_hbm, acc_s.at[pl.ds(s * STRIPE, STRIPE)])
    pltpu.sync_copy(dst_hbm.at[wid], dst_v)
    pltpu.sync_copy(ones_hbm, ones_v)
    plsc.subcore_barrier()

    def body(j, carry):
        pltpu.sync_copy(ones_v, acc_s.at[dst_v.at[j]], add=True)
        return carry

    lax.fori_loop(0, NCHUNK, body, 0)
    plsc.subcore_barrier()
    pltpu.sync_copy(acc_s.at[pl.ds(s * STRIPE, STRIPE)],
                    out_hbm.at[c, pl.ds(s * STRIPE, STRIPE)])


G = 16
NGROUP = NCHUNK // G


def _make_agg(D, mode="full"):
    @functools.partial(
        pl.kernel,
        out_type=jax.ShapeDtypeStruct((NC, ACC_ROWS, D), jnp.float32),
        mesh=_sc_mesh(),
        scratch_types=[
            pltpu.VMEM((G, K), jnp.int32),
            pltpu.VMEM((G, K), jnp.int32),
            pltpu.VMEM((K, D), jnp.float32),
            pltpu.VMEM((K, D), jnp.float32),
            pltpu.VMEM_SHARED((ACC_ROWS, D), jnp.float32),
            pltpu.SemaphoreType.DMA,
            pltpu.SemaphoreType.DMA,
        ],
        compiler_params=pltpu.CompilerParams(use_tc_tiling_on_sc=False),
    )
    def agg(h_hbm, src_hbm, dst_hbm, zeros_hbm, out_hbm,
            src_v, dst_v, rows0, rows1, acc_s, sem0, sem1):
        c = lax.axis_index("c")
        s = lax.axis_index("s")
        wid = c * NS + s
        pltpu.sync_copy(zeros_hbm, acc_s.at[pl.ds(s * STRIPE, STRIPE)])
        plsc.subcore_barrier()

        rows = (rows0, rows1)
        sems = (sem0, sem1)

        def group(g, carry):
            pltpu.sync_copy(src_hbm.at[wid, pl.ds(g * G, G)], src_v)
            pltpu.sync_copy(dst_hbm.at[wid, pl.ds(g * G, G)], dst_v)
            if mode != "scatter":
                for b in range(2):
                    pltpu.make_async_copy(h_hbm.at[src_v.at[b]], rows[b], sems[b]).start()

            def body(i, carry2):
                j = i * 2
                for b in range(2):
                    cj = j + b
                    if mode != "scatter":
                        pltpu.make_async_copy(h_hbm.at[src_v.at[cj]], rows[b], sems[b]).wait()
                    if mode != "gather":
                        pltpu.sync_copy(rows[b], acc_s.at[dst_v.at[cj]], add=True)
                    nj = cj + 2

                    if mode != "scatter":
                        @pl.when(nj < G)
                        def _():
                            pltpu.make_async_copy(h_hbm.at[src_v.at[nj]], rows[b], sems[b]).start()
                return carry2

            lax.fori_loop(0, G // 2, body, 0)
            return carry

        lax.fori_loop(0, NGROUP, group, 0)
        plsc.subcore_barrier()
        pltpu.sync_copy(acc_s.at[pl.ds(s * STRIPE, STRIPE)],
                        out_hbm.at[c, pl.ds(s * STRIPE, STRIPE)])

    return agg


_agg128 = _make_agg(F1)
_agg_gather = _make_agg(F1, "gather")
_agg_scatter = _make_agg(F1, "scatter")


def _dis_block(degp_ref):
    deg = degp_ref[0, :, 0:1] + degp_ref[1, :, 0:1] + 1.0
    return lax.rsqrt(deg)


def _mm1_body(degp_ref, x_ref, w_ref, out_ref):
    dis = _dis_block(degp_ref)
    h = jnp.dot(x_ref[...], w_ref[...], preferred_element_type=jnp.float32)
    out_ref[...] = h * dis


def _mid_body(degp_ref, a_ref, hs1_ref, b1_ref, out_ref):
    dis = _dis_block(degp_ref)
    h = dis * (a_ref[0] + a_ref[1] + hs1_ref[...]) + b1_ref[...]
    out_ref[...] = jnp.maximum(h, 0.0) * dis


def _fin_body(degp_ref, a_ref, u_ref, b2_ref, w2_ref, out_ref):
    dis = _dis_block(degp_ref)
    h = a_ref[0] + a_ref[1] + u_ref[...]
    out_ref[...] = dis * jnp.dot(h, w2_ref[...],
                                 preferred_element_type=jnp.float32) + b2_ref[...]


def _degp_spec():
    return pl.BlockSpec((NC, BM, 16), lambda i: (0, i, 0))


def _acc_spec(D):
    return pl.BlockSpec((NC, BM, D), lambda i: (0, i, 0))


_mm1 = pl.pallas_call(
    _mm1_body,
    grid=(N // BM,),
    in_specs=[
        _degp_spec(),
        pl.BlockSpec((BM, F1), lambda i: (i, 0)),
        pl.BlockSpec((F1, F1), lambda i: (0, 0)),
    ],
    out_specs=pl.BlockSpec((BM, F1), lambda i: (i, 0)),
    out_shape=jax.ShapeDtypeStruct((N, F1), jnp.float32),
)

_mid = pl.pallas_call(
    _mid_body,
    grid=(N // BM,),
    in_specs=[
        _degp_spec(),
        _acc_spec(F1),
        pl.BlockSpec((BM, F1), lambda i: (i, 0)),
        pl.BlockSpec((1, F1), lambda i: (0, 0)),
    ],
    out_specs=pl.BlockSpec((BM, F1), lambda i: (i, 0)),
    out_shape=jax.ShapeDtypeStruct((N, F1), jnp.float32),
)

_fin = pl.pallas_call(
    _fin_body,
    grid=(N // BM,),
    in_specs=[
        _degp_spec(),
        _acc_spec(F1),
        pl.BlockSpec((BM, F1), lambda i: (i, 0)),
        pl.BlockSpec((1, CP), lambda i: (0, 0)),
        pl.BlockSpec((F1, CP), lambda i: (0, 0)),
    ],
    out_specs=pl.BlockSpec((BM, CP), lambda i: (i, 0)),
    out_shape=jax.ShapeDtypeStruct((N, CP), jnp.float32),
)


@jax.jit
def _run(x, edge_index, W1, b1, W2, b2):
    src = edge_index[0].astype(jnp.int32)
    dst = edge_index[1].astype(jnp.int32)
    pad = E_PAD - E
    pad_dst = N + (jnp.arange(pad, dtype=jnp.int32) % (ACC_ROWS - N))
    src3 = jnp.concatenate([src, jnp.zeros((pad,), jnp.int32)]).reshape(NW, NCHUNK, K)
    dst3 = jnp.concatenate([dst, pad_dst]).reshape(NW, NCHUNK, K)

    ones16 = jnp.ones((K, 16), jnp.float32)
    z16 = jnp.zeros((STRIPE, 16), jnp.float32)
    z128 = jnp.zeros((STRIPE, F1), jnp.float32)
    W2p = jnp.pad(W2, ((0, 0), (0, CP - CLASSES)))
    b1r = b1.reshape(1, F1)
    b2r = jnp.pad(b2, (0, CP - CLASSES)).reshape(1, CP)

    degp = _deg_kernel(dst3, ones16, z16)
    hs1 = _mm1(degp, x, W1)
    acc1 = _agg128(hs1, src3, dst3, z128)
    u1 = _mid(degp, acc1, hs1, b1r)
    acc2 = _agg128(u1, src3, dst3, z128)
    u2 = _mid(degp, acc2, hs1, b1r)
    acc3 = _agg128(u2, src3, dst3, z128)
    u3 = _mid(degp, acc3, hs1, b1r)
    acc4 = _agg128(u3, src3, dst3, z128)
    outp = _fin(degp, acc4, u3, b2r, W2p)
    return outp[:, :CLASSES]


def kernel(x, edge_index, W1, b1, W2, b2):
    return _run(x, edge_index, W1, b1, W2, b2)


# speedup vs baseline: 1.7003x; 1.7003x over previous
import functools

import jax
import jax.numpy as jnp
from jax import lax
from jax.experimental import pallas as pl
from jax.experimental.pallas import tpu as pltpu
from jax.experimental.pallas import tpu_sc as plsc

N = 10000
E = 320000
F1 = 128
CP = 48
CLASSES = 40

NC, NS = 2, 16
NCU = 1
NW = NCU * NS
K = 128
E_PAD = 327680
NCHUNK = E_PAD // (NW * K)
assert NW * NCHUNK * K == E_PAD
ACC_ROWS = 10112
STRIPE = ACC_ROWS // NS

BM = 1000


def _sc_mesh():
    return plsc.VectorSubcoreMesh(
        core_axis_name="c", subcore_axis_name="s",
        num_cores=NCU, num_subcores=NS)


@functools.partial(
    pl.kernel,
    out_type=jax.ShapeDtypeStruct((NCU, ACC_ROWS, 16), jnp.float32),
    mesh=_sc_mesh(),
    scratch_types=[
        pltpu.VMEM((NCHUNK, K), jnp.int32),
        pltpu.VMEM((K, 16), jnp.float32),
        pltpu.VMEM_SHARED((ACC_ROWS, 16), jnp.float32),
    ],
    compiler_params=pltpu.CompilerParams(use_tc_tiling_on_sc=False),
)
def _deg_kernel(dst_hbm, ones_hbm, zeros_hbm, out_hbm, dst_v, ones_v, acc_s):
    c = lax.axis_index("c")
    s = lax.axis_index("s")
    wid = c * NS + s
    pltpu.sync_copy(zeros_hbm, acc_s.at[pl.ds(s * STRIPE, STRIPE)])
    pltpu.sync_copy(dst_hbm.at[wid], dst_v)
    pltpu.sync_copy(ones_hbm, ones_v)
    plsc.subcore_barrier()

    def body(j, carry):
        pltpu.sync_copy(ones_v, acc_s.at[dst_v.at[j]], add=True)
        return carry

    lax.fori_loop(0, NCHUNK, body, 0)
    plsc.subcore_barrier()
    pltpu.sync_copy(acc_s.at[pl.ds(s * STRIPE, STRIPE)],
                    out_hbm.at[c, pl.ds(s * STRIPE, STRIPE)])


G = 16
NGROUP = NCHUNK // G


def _make_agg(D, mode="full"):
    @functools.partial(
        pl.kernel,
        out_type=jax.ShapeDtypeStruct((NCU, ACC_ROWS, D), jnp.float32),
        mesh=_sc_mesh(),
        scratch_types=[
            pltpu.VMEM((G, K), jnp.int32),
            pltpu.VMEM((G, K), jnp.int32),
            pltpu.VMEM((K, D), jnp.float32),
            pltpu.VMEM((K, D), jnp.float32),
            pltpu.VMEM_SHARED((ACC_ROWS, D), jnp.float32),
            pltpu.SemaphoreType.DMA,
            pltpu.SemaphoreType.DMA,
        ],
        compiler_params=pltpu.CompilerParams(use_tc_tiling_on_sc=False),
    )
    def agg(h_hbm, src_hbm, dst_hbm, zeros_hbm, out_hbm,
            src_v, dst_v, rows0, rows1, acc_s, sem0, sem1):
        c = lax.axis_index("c")
        s = lax.axis_index("s")
        wid = c * NS + s
        pltpu.sync_copy(zeros_hbm, acc_s.at[pl.ds(s * STRIPE, STRIPE)])
        plsc.subcore_barrier()

        rows = (rows0, rows1)
        sems = (sem0, sem1)

        def group(g, carry):
            pltpu.sync_copy(src_hbm.at[wid, pl.ds(g * G, G)], src_v)
            pltpu.sync_copy(dst_hbm.at[wid, pl.ds(g * G, G)], dst_v)
            if mode != "scatter":
                for b in range(2):
                    pltpu.make_async_copy(h_hbm.at[src_v.at[b]], rows[b], sems[b]).start()

            def body(i, carry2):
                j = i * 2
                for b in range(2):
                    cj = j + b
                    if mode != "scatter":
                        pltpu.make_async_copy(h_hbm.at[src_v.at[cj]], rows[b], sems[b]).wait()
                    if mode != "gather":
                        pltpu.sync_copy(rows[b], acc_s.at[dst_v.at[cj]], add=True)
                    nj = cj + 2

                    if mode != "scatter":
                        @pl.when(nj < G)
                        def _():
                            pltpu.make_async_copy(h_hbm.at[src_v.at[nj]], rows[b], sems[b]).start()
                return carry2

            lax.fori_loop(0, G // 2, body, 0)
            return carry

        lax.fori_loop(0, NGROUP, group, 0)
        plsc.subcore_barrier()
        pltpu.sync_copy(acc_s.at[pl.ds(s * STRIPE, STRIPE)],
                        out_hbm.at[c, pl.ds(s * STRIPE, STRIPE)])

    return agg


_agg128 = _make_agg(F1)
_agg_gather = _make_agg(F1, "gather")
_agg_scatter = _make_agg(F1, "scatter")


def _dis_block(degp_ref):
    deg = degp_ref[...].sum(axis=0)[:, 0:1] + 1.0
    return lax.rsqrt(deg)


def _mm1_body(degp_ref, x_ref, w_ref, out_ref):
    dis = _dis_block(degp_ref)
    h = jnp.dot(x_ref[...], w_ref[...], preferred_element_type=jnp.float32)
    out_ref[...] = h * dis


def _mid_body(degp_ref, a_ref, hs1_ref, b1_ref, out_ref):
    dis = _dis_block(degp_ref)
    h = dis * (a_ref[...].sum(axis=0) + hs1_ref[...]) + b1_ref[...]
    out_ref[...] = jnp.maximum(h, 0.0) * dis


def _fin_body(degp_ref, a_ref, u_ref, b2_ref, w2_ref, out_ref):
    dis = _dis_block(degp_ref)
    h = a_ref[...].sum(axis=0) + u_ref[...]
    out_ref[...] = dis * jnp.dot(h, w2_ref[...],
                                 preferred_element_type=jnp.float32) + b2_ref[...]


def _degp_spec():
    return pl.BlockSpec((NCU, BM, 16), lambda i: (0, i, 0))


def _acc_spec(D):
    return pl.BlockSpec((NCU, BM, D), lambda i: (0, i, 0))


_mm1 = pl.pallas_call(
    _mm1_body,
    grid=(N // BM,),
    in_specs=[
        _degp_spec(),
        pl.BlockSpec((BM, F1), lambda i: (i, 0)),
        pl.BlockSpec((F1, F1), lambda i: (0, 0)),
    ],
    out_specs=pl.BlockSpec((BM, F1), lambda i: (i, 0)),
    out_shape=jax.ShapeDtypeStruct((N, F1), jnp.float32),
)

_mid = pl.pallas_call(
    _mid_body,
    grid=(N // BM,),
    in_specs=[
        _degp_spec(),
        _acc_spec(F1),
        pl.BlockSpec((BM, F1), lambda i: (i, 0)),
        pl.BlockSpec((1, F1), lambda i: (0, 0)),
    ],
    out_specs=pl.BlockSpec((BM, F1), lambda i: (i, 0)),
    out_shape=jax.ShapeDtypeStruct((N, F1), jnp.float32),
)

_fin = pl.pallas_call(
    _fin_body,
    grid=(N // BM,),
    in_specs=[
        _degp_spec(),
        _acc_spec(F1),
        pl.BlockSpec((BM, F1), lambda i: (i, 0)),
        pl.BlockSpec((1, CP), lambda i: (0, 0)),
        pl.BlockSpec((F1, CP), lambda i: (0, 0)),
    ],
    out_specs=pl.BlockSpec((BM, CP), lambda i: (i, 0)),
    out_shape=jax.ShapeDtypeStruct((N, CP), jnp.float32),
)


@jax.jit
def _run(x, edge_index, W1, b1, W2, b2):
    src = edge_index[0].astype(jnp.int32)
    dst = edge_index[1].astype(jnp.int32)
    pad = E_PAD - E
    pad_dst = N + (jnp.arange(pad, dtype=jnp.int32) % (ACC_ROWS - N))
    src3 = jnp.concatenate([src, jnp.zeros((pad,), jnp.int32)]).reshape(NW, NCHUNK, K)
    dst3 = jnp.concatenate([dst, pad_dst]).reshape(NW, NCHUNK, K)

    ones16 = jnp.ones((K, 16), jnp.float32)
    z16 = jnp.zeros((STRIPE, 16), jnp.float32)
    z128 = jnp.zeros((STRIPE, F1), jnp.float32)
    W2p = jnp.pad(W2, ((0, 0), (0, CP - CLASSES)))
    b1r = b1.reshape(1, F1)
    b2r = jnp.pad(b2, (0, CP - CLASSES)).reshape(1, CP)

    degp = _deg_kernel(dst3, ones16, z16)
    hs1 = _mm1(degp, x, W1)
    acc1 = _agg128(hs1, src3, dst3, z128)
    u = _mid(degp, acc1, hs1, b1r)
    acc2 = _agg128(u, src3, dst3, z128)
    outp = _fin(degp, acc2, u, b2r, W2p)
    return outp[:, :CLASSES]


def kernel(x, edge_index, W1, b1, W2, b2):
    return _run(x, edge_index, W1, b1, W2, b2)


# speedup vs baseline: 1.7194x; 1.0112x over previous
import functools

import jax
import jax.numpy as jnp
from jax import lax
from jax.experimental import pallas as pl
from jax.experimental.pallas import tpu as pltpu
from jax.experimental.pallas import tpu_sc as plsc

N = 10000
E = 320000
F1 = 128
CP = 48
CLASSES = 40

NC, NS = 2, 16
NCU = 2
NW = NCU * NS
K = 80
E_PAD = 327680
NCHUNK = E_PAD // (NW * K)
assert NW * NCHUNK * K == E_PAD
ACC_ROWS = 10112
STRIPE = ACC_ROWS // NS

BM = 1000


def _sc_mesh():
    return plsc.VectorSubcoreMesh(
        core_axis_name="c", subcore_axis_name="s",
        num_cores=NCU, num_subcores=NS)


@functools.partial(
    pl.kernel,
    out_type=jax.ShapeDtypeStruct((NCU, ACC_ROWS, 16), jnp.float32),
    mesh=_sc_mesh(),
    scratch_types=[
        pltpu.VMEM((NCHUNK, K), jnp.int32),
        pltpu.VMEM((K, 16), jnp.float32),
        pltpu.VMEM_SHARED((ACC_ROWS, 16), jnp.float32),
    ],
    compiler_params=pltpu.CompilerParams(use_tc_tiling_on_sc=False),
)
def _deg_kernel(dst_hbm, ones_hbm, zeros_hbm, out_hbm, dst_v, ones_v, acc_s):
    c = lax.axis_index("c")
    s = lax.axis_index("s")
    wid = c * NS + s
    pltpu.sync_copy(zeros_hbm, acc_s.at[pl.ds(s * STRIPE, STRIPE)])
    pltpu.sync_copy(dst_hbm.at[wid], dst_v)
    pltpu.sync_copy(ones_hbm, ones_v)
    plsc.subcore_barrier()

    def body(j, carry):
        pltpu.sync_copy(ones_v, acc_s.at[dst_v.at[j]], add=True)
        return carry

    lax.fori_loop(0, NCHUNK, body, 0)
    plsc.subcore_barrier()
    pltpu.sync_copy(acc_s.at[pl.ds(s * STRIPE, STRIPE)],
                    out_hbm.at[c, pl.ds(s * STRIPE, STRIPE)])


G = 16
NGROUP = NCHUNK // G
NBUF = 4


def _make_agg(D):
    @functools.partial(
        pl.kernel,
        out_type=jax.ShapeDtypeStruct((NCU, ACC_ROWS, D), jnp.float32),
        mesh=_sc_mesh(),
        scratch_types=(
            [pltpu.VMEM((G, K), jnp.int32),
             pltpu.VMEM((G, K), jnp.int32)]
            + [pltpu.VMEM((K, D), jnp.float32) for _ in range(NBUF)]
            + [pltpu.VMEM_SHARED((ACC_ROWS, D), jnp.float32)]
            + [pltpu.SemaphoreType.DMA for _ in range(2 * NBUF)]
        ),
        compiler_params=pltpu.CompilerParams(use_tc_tiling_on_sc=False),
    )
    def agg(h_hbm, src_hbm, dst_hbm, zeros_hbm, out_hbm,
            src_v, dst_v, *rest):
        rows = rest[:NBUF]
        acc_s = rest[NBUF]
        gsem = rest[NBUF + 1:NBUF + 1 + NBUF]
        ssem = rest[NBUF + 1 + NBUF:]
        c = lax.axis_index("c")
        s = lax.axis_index("s")
        wid = c * NS + s
        pltpu.sync_copy(zeros_hbm, acc_s.at[pl.ds(s * STRIPE, STRIPE)])
        plsc.subcore_barrier()

        def group(g, carry):
            pltpu.sync_copy(src_hbm.at[wid, pl.ds(g * G, G)], src_v)
            pltpu.sync_copy(dst_hbm.at[wid, pl.ds(g * G, G)], dst_v)
            for b in range(NBUF):
                pltpu.make_async_copy(h_hbm.at[src_v.at[b]], rows[b], gsem[b]).start()

            def body(i, carry2):
                j = i * NBUF
                for b in range(NBUF):
                    pltpu.make_async_copy(h_hbm.at[src_v.at[j + b]], rows[b], gsem[b]).wait()
                    pltpu.make_async_copy(rows[b], acc_s.at[dst_v.at[j + b]], ssem[b]).start(add=True)
                for b in range(NBUF):
                    nj = j + NBUF + b

                    @pl.when(nj < G)
                    def _():
                        pltpu.make_async_copy(rows[b], acc_s.at[dst_v.at[j + b]], ssem[b]).wait()
                        pltpu.make_async_copy(h_hbm.at[src_v.at[nj]], rows[b], gsem[b]).start()
                return carry2

            lax.fori_loop(0, G // NBUF, body, 0)
            for b in range(NBUF):
                pltpu.make_async_copy(rows[b], acc_s.at[dst_v.at[G - NBUF + b]], ssem[b]).wait()
            return carry

        lax.fori_loop(0, NGROUP, group, 0)
        plsc.subcore_barrier()
        pltpu.sync_copy(acc_s.at[pl.ds(s * STRIPE, STRIPE)],
                        out_hbm.at[c, pl.ds(s * STRIPE, STRIPE)])

    return agg


_agg128 = _make_agg(F1)


def _dis_block(degp_ref):
    deg = degp_ref[...].sum(axis=0)[:, 0:1] + 1.0
    return lax.rsqrt(deg)


def _mm1_body(degp_ref, x_ref, w_ref, out_ref):
    dis = _dis_block(degp_ref)
    h = jnp.dot(x_ref[...], w_ref[...], preferred_element_type=jnp.float32)
    out_ref[...] = h * dis


def _mid_body(degp_ref, a_ref, hs1_ref, b1_ref, out_ref):
    dis = _dis_block(degp_ref)
    h = dis * (a_ref[...].sum(axis=0) + hs1_ref[...]) + b1_ref[...]
    out_ref[...] = jnp.maximum(h, 0.0) * dis


def _fin_body(degp_ref, a_ref, u_ref, b2_ref, w2_ref, out_ref):
    dis = _dis_block(degp_ref)
    h = a_ref[...].sum(axis=0) + u_ref[...]
    out_ref[...] = dis * jnp.dot(h, w2_ref[...],
                                 preferred_element_type=jnp.float32) + b2_ref[...]


def _degp_spec():
    return pl.BlockSpec((NCU, BM, 16), lambda i: (0, i, 0))


def _acc_spec(D):
    return pl.BlockSpec((NCU, BM, D), lambda i: (0, i, 0))


_mm1 = pl.pallas_call(
    _mm1_body,
    grid=(N // BM,),
    in_specs=[
        _degp_spec(),
        pl.BlockSpec((BM, F1), lambda i: (i, 0)),
        pl.BlockSpec((F1, F1), lambda i: (0, 0)),
    ],
    out_specs=pl.BlockSpec((BM, F1), lambda i: (i, 0)),
    out_shape=jax.ShapeDtypeStruct((N, F1), jnp.float32),
)

_mid = pl.pallas_call(
    _mid_body,
    grid=(N // BM,),
    in_specs=[
        _degp_spec(),
        _acc_spec(F1),
        pl.BlockSpec((BM, F1), lambda i: (i, 0)),
        pl.BlockSpec((1, F1), lambda i: (0, 0)),
    ],
    out_specs=pl.BlockSpec((BM, F1), lambda i: (i, 0)),
    out_shape=jax.ShapeDtypeStruct((N, F1), jnp.float32),
)

_fin = pl.pallas_call(
    _fin_body,
    grid=(N // BM,),
    in_specs=[
        _degp_spec(),
        _acc_spec(F1),
        pl.BlockSpec((BM, F1), lambda i: (i, 0)),
        pl.BlockSpec((1, CP), lambda i: (0, 0)),
        pl.BlockSpec((F1, CP), lambda i: (0, 0)),
    ],
    out_specs=pl.BlockSpec((BM, CP), lambda i: (i, 0)),
    out_shape=jax.ShapeDtypeStruct((N, CP), jnp.float32),
)


@jax.jit
def _run(x, edge_index, W1, b1, W2, b2):
    src = edge_index[0].astype(jnp.int32)
    dst = edge_index[1].astype(jnp.int32)
    pad = E_PAD - E
    pad_dst = N + (jnp.arange(pad, dtype=jnp.int32) % (ACC_ROWS - N))
    src3 = jnp.concatenate([src, jnp.zeros((pad,), jnp.int32)]).reshape(NW, NCHUNK, K)
    dst3 = jnp.concatenate([dst, pad_dst]).reshape(NW, NCHUNK, K)

    ones16 = jnp.ones((K, 16), jnp.float32)
    z16 = jnp.zeros((STRIPE, 16), jnp.float32)
    z128 = jnp.zeros((STRIPE, F1), jnp.float32)
    W2p = jnp.pad(W2, ((0, 0), (0, CP - CLASSES)))
    b1r = b1.reshape(1, F1)
    b2r = jnp.pad(b2, (0, CP - CLASSES)).reshape(1, CP)

    degp = _deg_kernel(dst3, ones16, z16)
    hs1 = _mm1(degp, x, W1)
    acc1 = _agg128(hs1, src3, dst3, z128)
    u = _mid(degp, acc1, hs1, b1r)
    acc2 = _agg128(u, src3, dst3, z128)
    outp = _fin(degp, acc2, u, b2r, W2p)
    return outp[:, :CLASSES]


def kernel(x, edge_index, W1, b1, W2, b2):
    return _run(x, edge_index, W1, b1, W2, b2)


# speedup vs baseline: 1.7249x; 1.0032x over previous
import functools

import jax
import jax.numpy as jnp
from jax import lax
from jax.experimental import pallas as pl
from jax.experimental.pallas import tpu as pltpu
from jax.experimental.pallas import tpu_sc as plsc

N = 10000
E = 320000
F1 = 128
CP = 48
CLASSES = 40

NC, NS = 2, 16
NCU = 2
NW = NCU * NS
K = 128
E_PAD = 327680
NCHUNK = E_PAD // (NW * K)
assert NW * NCHUNK * K == E_PAD
ACC_ROWS = 10112
STRIPE = ACC_ROWS // NS

BM = 1000


def _sc_mesh():
    return plsc.VectorSubcoreMesh(
        core_axis_name="c", subcore_axis_name="s",
        num_cores=NCU, num_subcores=NS)


@functools.partial(
    pl.kernel,
    out_type=jax.ShapeDtypeStruct((NCU, ACC_ROWS, 16), jnp.float32),
    mesh=_sc_mesh(),
    scratch_types=[
        pltpu.VMEM((NCHUNK, K), jnp.int32),
        pltpu.VMEM((K, 16), jnp.float32),
        pltpu.VMEM_SHARED((ACC_ROWS, 16), jnp.float32),
    ],
    compiler_params=pltpu.CompilerParams(use_tc_tiling_on_sc=False),
)
def _deg_kernel(dst_hbm, ones_hbm, zeros_hbm, out_hbm, dst_v, ones_v, acc_s):
    c = lax.axis_index("c")
    s = lax.axis_index("s")
    wid = c * NS + s
    pltpu.sync_copy(zeros_hbm, acc_s.at[pl.ds(s * STRIPE, STRIPE)])
    pltpu.sync_copy(dst_hbm.at[wid], dst_v)
    pltpu.sync_copy(ones_hbm, ones_v)
    plsc.subcore_barrier()

    def body(j, carry):
        pltpu.sync_copy(ones_v, acc_s.at[dst_v.at[j]], add=True)
        return carry

    lax.fori_loop(0, NCHUNK, body, 0)
    plsc.subcore_barrier()
    pltpu.sync_copy(acc_s.at[pl.ds(s * STRIPE, STRIPE)],
                    out_hbm.at[c, pl.ds(s * STRIPE, STRIPE)])


G = 16
NGROUP = NCHUNK // G
NBUF = 2


def _make_agg(D):
    @functools.partial(
        pl.kernel,
        out_type=jax.ShapeDtypeStruct((NCU, ACC_ROWS, D), jnp.float32),
        mesh=_sc_mesh(),
        scratch_types=(
            [pltpu.VMEM((G, K), jnp.int32),
             pltpu.VMEM((G, K), jnp.int32)]
            + [pltpu.VMEM((K, D), jnp.float32) for _ in range(NBUF)]
            + [pltpu.VMEM_SHARED((ACC_ROWS, D), jnp.float32)]
            + [pltpu.SemaphoreType.DMA for _ in range(2 * NBUF)]
        ),
        compiler_params=pltpu.CompilerParams(use_tc_tiling_on_sc=False),
    )
    def agg(h_hbm, src_hbm, dst_hbm, zeros_hbm, out_hbm,
            src_v, dst_v, *rest):
        rows = rest[:NBUF]
        acc_s = rest[NBUF]
        gsem = rest[NBUF + 1:NBUF + 1 + NBUF]
        ssem = rest[NBUF + 1 + NBUF:]
        c = lax.axis_index("c")
        s = lax.axis_index("s")
        wid = c * NS + s
        pltpu.sync_copy(zeros_hbm, acc_s.at[pl.ds(s * STRIPE, STRIPE)])
        plsc.subcore_barrier()

        def group(g, carry):
            pltpu.sync_copy(src_hbm.at[wid, pl.ds(g * G, G)], src_v)
            pltpu.sync_copy(dst_hbm.at[wid, pl.ds(g * G, G)], dst_v)
            for b in range(NBUF):
                pltpu.make_async_copy(h_hbm.at[src_v.at[b]], rows[b], gsem[b]).start()

            def body(i, carry2):
                j = i * NBUF
                for b in range(NBUF):
                    pltpu.make_async_copy(h_hbm.at[src_v.at[j + b]], rows[b], gsem[b]).wait()
                    pltpu.make_async_copy(rows[b], acc_s.at[dst_v.at[j + b]], ssem[b]).start(add=True)
                for b in range(NBUF):
                    nj = j + NBUF + b

                    @pl.when(nj < G)
                    def _():
                        pltpu.make_async_copy(rows[b], acc_s.at[dst_v.at[j + b]], ssem[b]).wait()
                        pltpu.make_async_copy(h_hbm.at[src_v.at[nj]], rows[b], gsem[b]).start()
                return carry2

            lax.fori_loop(0, G // NBUF, body, 0)
            for b in range(NBUF):
                pltpu.make_async_copy(rows[b], acc_s.at[dst_v.at[G - NBUF + b]], ssem[b]).wait()
            return carry

        lax.fori_loop(0, NGROUP, group, 0)
        plsc.subcore_barrier()
        pltpu.sync_copy(acc_s.at[pl.ds(s * STRIPE, STRIPE)],
                        out_hbm.at[c, pl.ds(s * STRIPE, STRIPE)])

    return agg


_agg128 = _make_agg(F1)


def _dis_block(degp_ref):
    deg = degp_ref[...].sum(axis=0)[:, 0:1] + 1.0
    return lax.rsqrt(deg)


def _mm1_body(degp_ref, x_ref, w_ref, out_ref):
    dis = _dis_block(degp_ref)
    h = jnp.dot(x_ref[...], w_ref[...], preferred_element_type=jnp.float32)
    out_ref[...] = h * dis


def _mid_body(degp_ref, a_ref, hs1_ref, b1_ref, out_ref):
    dis = _dis_block(degp_ref)
    h = dis * (a_ref[...].sum(axis=0) + hs1_ref[...]) + b1_ref[...]
    out_ref[...] = jnp.maximum(h, 0.0) * dis


def _fin_body(degp_ref, a_ref, u_ref, b2_ref, w2_ref, out_ref):
    dis = _dis_block(degp_ref)
    h = a_ref[...].sum(axis=0) + u_ref[...]
    out_ref[...] = dis * jnp.dot(h, w2_ref[...],
                                 preferred_element_type=jnp.float32) + b2_ref[...]


def _degp_spec():
    return pl.BlockSpec((NCU, BM, 16), lambda i: (0, i, 0))


def _acc_spec(D):
    return pl.BlockSpec((NCU, BM, D), lambda i: (0, i, 0))


_mm1 = pl.pallas_call(
    _mm1_body,
    grid=(N // BM,),
    in_specs=[
        _degp_spec(),
        pl.BlockSpec((BM, F1), lambda i: (i, 0)),
        pl.BlockSpec((F1, F1), lambda i: (0, 0)),
    ],
    out_specs=pl.BlockSpec((BM, F1), lambda i: (i, 0)),
    out_shape=jax.ShapeDtypeStruct((N, F1), jnp.float32),
)

_mid = pl.pallas_call(
    _mid_body,
    grid=(N // BM,),
    in_specs=[
        _degp_spec(),
        _acc_spec(F1),
        pl.BlockSpec((BM, F1), lambda i: (i, 0)),
        pl.BlockSpec((1, F1), lambda i: (0, 0)),
    ],
    out_specs=pl.BlockSpec((BM, F1), lambda i: (i, 0)),
    out_shape=jax.ShapeDtypeStruct((N, F1), jnp.float32),
)

_fin = pl.pallas_call(
    _fin_body,
    grid=(N // BM,),
    in_specs=[
        _degp_spec(),
        _acc_spec(F1),
        pl.BlockSpec((BM, F1), lambda i: (i, 0)),
        pl.BlockSpec((1, CP), lambda i: (0, 0)),
        pl.BlockSpec((F1, CP), lambda i: (0, 0)),
    ],
    out_specs=pl.BlockSpec((BM, CP), lambda i: (i, 0)),
    out_shape=jax.ShapeDtypeStruct((N, CP), jnp.float32),
)


@jax.jit
def _run(x, edge_index, W1, b1, W2, b2):
    src = edge_index[0].astype(jnp.int32)
    dst = edge_index[1].astype(jnp.int32)
    pad = E_PAD - E
    pad_dst = N + (jnp.arange(pad, dtype=jnp.int32) % (ACC_ROWS - N))
    src3 = jnp.concatenate([src, jnp.zeros((pad,), jnp.int32)]).reshape(NW, NCHUNK, K)
    dst3 = jnp.concatenate([dst, pad_dst]).reshape(NW, NCHUNK, K)

    ones16 = jnp.ones((K, 16), jnp.float32)
    z16 = jnp.zeros((STRIPE, 16), jnp.float32)
    z128 = jnp.zeros((STRIPE, F1), jnp.float32)
    W2p = jnp.pad(W2, ((0, 0), (0, CP - CLASSES)))
    b1r = b1.reshape(1, F1)
    b2r = jnp.pad(b2, (0, CP - CLASSES)).reshape(1, CP)

    degp = _deg_kernel(dst3, ones16, z16)
    hs1 = _mm1(degp, x, W1)
    acc1 = _agg128(hs1, src3, dst3, z128)
    u = _mid(degp, acc1, hs1, b1r)
    acc2 = _agg128(u, src3, dst3, z128)
    outp = _fin(degp, acc2, u, b2r, W2p)
    return outp[:, :CLASSES]


def kernel(x, edge_index, W1, b1, W2, b2):
    return _run(x, edge_index, W1, b1, W2, b2)


# speedup vs baseline: 1.8975x; 1.1001x over previous
import functools

import jax
import jax.numpy as jnp
from jax import lax
from jax.experimental import pallas as pl
from jax.experimental.pallas import tpu as pltpu
from jax.experimental.pallas import tpu_sc as plsc

N = 10000
E = 320000
F1 = 128
CP = 48
CLASSES = 40

NC, NS = 2, 16
NCU = 2
NW = NCU * NS
K = 128
E_PAD = 327680
NCHUNK = E_PAD // (NW * K)
assert NW * NCHUNK * K == E_PAD
ACC_ROWS = 10112
STRIPE = ACC_ROWS // NS

BM = 1000


def _sc_mesh():
    return plsc.VectorSubcoreMesh(
        core_axis_name="c", subcore_axis_name="s",
        num_cores=NCU, num_subcores=NS)


@functools.cache
def _get_deg_kernel():
    @functools.partial(
        pl.kernel,
        out_type=jax.ShapeDtypeStruct((NCU, ACC_ROWS, 16), jnp.float32),
        mesh=_sc_mesh(),
        scratch_types=[
            pltpu.VMEM((NCHUNK, K), jnp.int32),
            pltpu.VMEM((K, 16), jnp.float32),
            pltpu.VMEM_SHARED((ACC_ROWS, 16), jnp.float32),
        ],
        compiler_params=pltpu.CompilerParams(use_tc_tiling_on_sc=False),
    )
    def _deg_kernel(dst_hbm, ones_hbm, zeros_hbm, out_hbm, dst_v, ones_v, acc_s):
        c = lax.axis_index("c")
        s = lax.axis_index("s")
        wid = c * NS + s
        pltpu.sync_copy(zeros_hbm, acc_s.at[pl.ds(s * STRIPE, STRIPE)])
        pltpu.sync_copy(dst_hbm.at[wid], dst_v)
        pltpu.sync_copy(ones_hbm, ones_v)
        plsc.subcore_barrier()

        def body(j, carry):
            pltpu.sync_copy(ones_v, acc_s.at[dst_v.at[j]], add=True)
            return carry

        lax.fori_loop(0, NCHUNK, body, 0)
        plsc.subcore_barrier()
        pltpu.sync_copy(acc_s.at[pl.ds(s * STRIPE, STRIPE)],
                        out_hbm.at[c, pl.ds(s * STRIPE, STRIPE)])


    return _deg_kernel


G = 8
NBUF = 2
NCH0, NCH1 = 120, 40
TOT_CHUNKS = NS * (NCH0 + NCH1)
assert TOT_CHUNKS * K == E_PAD and NCH0 % G == 0 and NCH1 % G == 0


@functools.cache
def _make_agg(D):
    @functools.partial(
        pl.kernel,
        out_type=jax.ShapeDtypeStruct((NCU, ACC_ROWS, D), jnp.float32),
        mesh=_sc_mesh(),
        scratch_types=(
            [pltpu.VMEM((G, K), jnp.int32),
             pltpu.VMEM((G, K), jnp.int32)]
            + [pltpu.VMEM((K, D), jnp.float32) for _ in range(NBUF)]
            + [pltpu.VMEM_SHARED((ACC_ROWS, D), jnp.float32)]
            + [pltpu.SemaphoreType.DMA for _ in range(2 * NBUF)]
        ),
        compiler_params=pltpu.CompilerParams(use_tc_tiling_on_sc=False),
    )
    def agg(h_hbm, src_hbm, dst_hbm, zeros_hbm, out_hbm,
            src_v, dst_v, *rest):
        rows = rest[:NBUF]
        acc_s = rest[NBUF]
        gsem = rest[NBUF + 1:NBUF + 1 + NBUF]
        ssem = rest[NBUF + 1 + NBUF:]
        c = lax.axis_index("c")
        s = lax.axis_index("s")
        cbase = jnp.where(c == 0, s * NCH0, NS * NCH0 + s * NCH1)
        ngroup = jnp.where(c == 0, NCH0 // G, NCH1 // G)
        pltpu.sync_copy(zeros_hbm, acc_s.at[pl.ds(s * STRIPE, STRIPE)])
        plsc.subcore_barrier()

        def group(g, carry):
            gb = cbase + g * G
            pltpu.sync_copy(src_hbm.at[pl.ds(gb, G)], src_v)
            pltpu.sync_copy(dst_hbm.at[pl.ds(gb, G)], dst_v)
            for b in range(NBUF):
                pltpu.make_async_copy(h_hbm.at[src_v.at[b]], rows[b], gsem[b]).start()

            def body(i, carry2):
                j = i * NBUF
                for b in range(NBUF):
                    pltpu.make_async_copy(h_hbm.at[src_v.at[j + b]], rows[b], gsem[b]).wait()
                    pltpu.make_async_copy(rows[b], acc_s.at[dst_v.at[j + b]], ssem[b]).start(add=True)
                for b in range(NBUF):
                    nj = j + NBUF + b

                    @pl.when(nj < G)
                    def _():
                        pltpu.make_async_copy(rows[b], acc_s.at[dst_v.at[j + b]], ssem[b]).wait()
                        pltpu.make_async_copy(h_hbm.at[src_v.at[nj]], rows[b], gsem[b]).start()
                return carry2

            lax.fori_loop(0, G // NBUF, body, 0)
            for b in range(NBUF):
                pltpu.make_async_copy(rows[b], acc_s.at[dst_v.at[G - NBUF + b]], ssem[b]).wait()
            return carry

        lax.fori_loop(0, ngroup, group, 0)
        plsc.subcore_barrier()
        pltpu.sync_copy(acc_s.at[pl.ds(s * STRIPE, STRIPE)],
                        out_hbm.at[c, pl.ds(s * STRIPE, STRIPE)])

    return agg


def _dis_block(degp_ref):
    deg = degp_ref[...].sum(axis=0)[:, 0:1] + 1.0
    return lax.rsqrt(deg)


def _mm1_body(degp_ref, x_ref, w_ref, out_ref):
    dis = _dis_block(degp_ref)
    h = jnp.dot(x_ref[...], w_ref[...], preferred_element_type=jnp.float32)
    out_ref[...] = h * dis


def _mid_body(degp_ref, a_ref, hs1_ref, b1_ref, out_ref):
    dis = _dis_block(degp_ref)
    h = dis * (a_ref[...].sum(axis=0) + hs1_ref[...]) + b1_ref[...]
    out_ref[...] = jnp.maximum(h, 0.0) * dis


def _fin_body(degp_ref, a_ref, u_ref, b2_ref, w2_ref, out_ref):
    dis = _dis_block(degp_ref)
    h = a_ref[...].sum(axis=0) + u_ref[...]
    out_ref[...] = dis * jnp.dot(h, w2_ref[...],
                                 preferred_element_type=jnp.float32) + b2_ref[...]


def _degp_spec():
    return pl.BlockSpec((NCU, BM, 16), lambda i: (0, i, 0))


def _acc_spec(D):
    return pl.BlockSpec((NCU, BM, D), lambda i: (0, i, 0))


_mm1 = pl.pallas_call(
    _mm1_body,
    grid=(N // BM,),
    in_specs=[
        _degp_spec(),
        pl.BlockSpec((BM, F1), lambda i: (i, 0)),
        pl.BlockSpec((F1, F1), lambda i: (0, 0)),
    ],
    out_specs=pl.BlockSpec((BM, F1), lambda i: (i, 0)),
    out_shape=jax.ShapeDtypeStruct((N, F1), jnp.float32),
)

_mid = pl.pallas_call(
    _mid_body,
    grid=(N // BM,),
    in_specs=[
        _degp_spec(),
        _acc_spec(F1),
        pl.BlockSpec((BM, F1), lambda i: (i, 0)),
        pl.BlockSpec((1, F1), lambda i: (0, 0)),
    ],
    out_specs=pl.BlockSpec((BM, F1), lambda i: (i, 0)),
    out_shape=jax.ShapeDtypeStruct((N, F1), jnp.float32),
)

_fin = pl.pallas_call(
    _fin_body,
    grid=(N // BM,),
    in_specs=[
        _degp_spec(),
        _acc_spec(F1),
        pl.BlockSpec((BM, F1), lambda i: (i, 0)),
        pl.BlockSpec((1, CP), lambda i: (0, 0)),
        pl.BlockSpec((F1, CP), lambda i: (0, 0)),
    ],
    out_specs=pl.BlockSpec((BM, CP), lambda i: (i, 0)),
    out_shape=jax.ShapeDtypeStruct((N, CP), jnp.float32),
)


@jax.jit
def _run(x, edge_index, W1, b1, W2, b2):
    src = edge_index[0].astype(jnp.int32)
    dst = edge_index[1].astype(jnp.int32)
    pad = E_PAD - E
    pad_dst = N + (jnp.arange(pad, dtype=jnp.int32) % (ACC_ROWS - N))
    src_p = jnp.concatenate([src, jnp.zeros((pad,), jnp.int32)])
    dst_p = jnp.concatenate([dst, pad_dst])
    src3 = src_p.reshape(NW, NCHUNK, K)
    dst3 = dst_p.reshape(NW, NCHUNK, K)
    srcF = src_p.reshape(TOT_CHUNKS, K)
    dstF = dst_p.reshape(TOT_CHUNKS, K)

    ones16 = jnp.ones((K, 16), jnp.float32)
    z16 = jnp.zeros((STRIPE, 16), jnp.float32)
    z128 = jnp.zeros((STRIPE, F1), jnp.float32)
    W2p = jnp.pad(W2, ((0, 0), (0, CP - CLASSES)))
    b1r = b1.reshape(1, F1)
    b2r = jnp.pad(b2, (0, CP - CLASSES)).reshape(1, CP)

    degp = _get_deg_kernel()(dst3, ones16, z16)
    hs1 = _mm1(degp, x, W1)
    acc1 = _make_agg(F1)(hs1, srcF, dstF, z128)
    u = _mid(degp, acc1, hs1, b1r)
    acc2 = _make_agg(F1)(u, srcF, dstF, z128)
    outp = _fin(degp, acc2, u, b2r, W2p)
    return outp[:, :CLASSES]


def kernel(x, edge_index, W1, b1, W2, b2):
    return _run(x, edge_index, W1, b1, W2, b2)
